# baseline (speedup 1.0000x reference)
import jax
import jax.numpy as jnp
from jax import lax
from jax.experimental import pallas as pl
from jax.experimental.pallas import tpu as pltpu
from jax.experimental.pallas import tpu_sc as plsc

B, S, D, H, E = 1, 2048, 1024, 4096, 8
T = B * S
THRESHOLD = 0.5

HC = 4
HB = H // HC
TBS = 256
NTB = T // TBS

NC, NS = 2, 16
NW = NC * NS
TPW = T // NW

_mesh = plsc.VectorSubcoreMesh(core_axis_name="c", subcore_axis_name="s")


def _sc_scatter_kernel(xb_hbm, posa_hbm, xg_hbm, rows_v, idx_v):
    wid = lax.axis_index("s") * NC + lax.axis_index("c")
    tok0 = wid * TPW
    pltpu.sync_copy(xb_hbm.at[pl.ds(tok0, TPW)], rows_v)
    for e in range(E):
        pltpu.sync_copy(posa_hbm.at[e, pl.ds(tok0, TPW)], idx_v)
        pltpu.sync_copy(rows_v, xg_hbm.at[idx_v])


def _sc_scatter(xb3, posa):
    return pl.kernel(
        _sc_scatter_kernel,
        out_type=jax.ShapeDtypeStruct((E * T, 4, 128), jnp.int32),
        mesh=_mesh,
        scratch_types=[
            pltpu.VMEM((TPW, 4, 128), jnp.int32),
            pltpu.VMEM((TPW,), jnp.int32),
        ],
    )(xb3, posa)


def _sc_gather_kernel(yg_hbm, posb_hbm, yd_hbm, rows_v, idx_v):
    wid = lax.axis_index("s") * NC + lax.axis_index("c")
    tok0 = wid * TPW
    for e in range(E):
        pltpu.sync_copy(posb_hbm.at[e, pl.ds(tok0, TPW)], idx_v)
        pltpu.sync_copy(yg_hbm.at[idx_v], rows_v)
        pltpu.sync_copy(rows_v, yd_hbm.at[pl.ds(e * T + tok0, TPW)])


def _sc_gather(yg3, posb):
    return pl.kernel(
        _sc_gather_kernel,
        out_type=jax.ShapeDtypeStruct((E * T, 4, 128), jnp.int32),
        mesh=_mesh,
        scratch_types=[
            pltpu.VMEM((TPW, 4, 128), jnp.int32),
            pltpu.VMEM((TPW,), jnp.int32),
        ],
    )(yg3, posb)


def _ffn_kernel(cnt_ref, xg_ref, w1_ref, b1_ref, w2_ref, b2_ref, yg_ref,
                wc1_ref, wc2_ref, yacc_ref):
    e = pl.program_id(0)
    hc = pl.program_id(1)
    tb = pl.program_id(2)

    @pl.when(tb == 0)
    def _cache_weights():
        wc1_ref[...] = w1_ref[0].astype(jnp.bfloat16)
        wc2_ref[...] = w2_ref[0].astype(jnp.bfloat16)

    @pl.when(tb * TBS < cnt_ref[e])
    def _compute():
        xb = xg_ref[0]
        h = jax.lax.dot_general(xb, wc1_ref[...], (((1,), (0,)), ((), ())),
                                preferred_element_type=jnp.float32)
        hb = jnp.maximum(h + b1_ref[0], 0.0).astype(jnp.bfloat16)
        y = jax.lax.dot_general(hb, wc2_ref[...], (((1,), (0,)), ((), ())),
                                preferred_element_type=jnp.float32)
        y = y + jnp.where(hc == 0, 1.0, 0.0) * b2_ref[0]

        @pl.when(hc == 0)
        def _first():
            yacc_ref[...] = y

        @pl.when(hc != 0)
        def _rest():
            yacc_ref[...] += y

        @pl.when(hc == HC - 1)
        def _emit():
            yg_ref[0] = yacc_ref[...].astype(jnp.bfloat16)


def _ffn(counts, xg, W1, b1r, W2, b2r):
    return pl.pallas_call(
        _ffn_kernel,
        grid=(E, HC, NTB),
        in_specs=[
            pl.BlockSpec(memory_space=pltpu.SMEM),
            pl.BlockSpec((1, TBS, D), lambda e, hc, tb: (e, tb, 0)),
            pl.BlockSpec((1, D, HB), lambda e, hc, tb: (e, 0, hc)),
            pl.BlockSpec((1, 1, HB), lambda e, hc, tb: (e, 0, hc)),
            pl.BlockSpec((1, HB, D), lambda e, hc, tb: (e, hc, 0)),
            pl.BlockSpec((1, 1, D), lambda e, hc, tb: (e, 0, 0)),
        ],
        out_specs=pl.BlockSpec((1, TBS, D), lambda e, hc, tb: (e, tb, 0)),
        out_shape=jax.ShapeDtypeStruct((E, T, D), jnp.bfloat16),
        scratch_shapes=[
            pltpu.VMEM((D, HB), jnp.bfloat16),
            pltpu.VMEM((HB, D), jnp.bfloat16),
            pltpu.VMEM((TBS, D), jnp.float32),
        ],
        compiler_params=pltpu.CompilerParams(
            dimension_semantics=("arbitrary", "arbitrary", "arbitrary")),
    )(counts, xg, W1, b1r, W2, b2r)


def _combine_kernel(ew_ref, yd_ref, out_ref):
    e = pl.program_id(0)

    @pl.when(e == 0)
    def _init():
        out_ref[...] = jnp.zeros((T, D), jnp.float32)

    ewb = ew_ref[...]
    lane = jax.lax.broadcasted_iota(jnp.int32, (T, E), 1)
    w = jnp.sum(jnp.where(lane == e, ewb, 0.0), axis=1, keepdims=True)
    y = yd_ref[0].astype(jnp.float32)
    out_ref[...] += jnp.where(w > 0.0, w * y, 0.0)


def _combine(ew, yd):
    return pl.pallas_call(
        _combine_kernel,
        grid=(E,),
        in_specs=[
            pl.BlockSpec((T, E), lambda e: (0, 0)),
            pl.BlockSpec((1, T, D), lambda e: (e, 0, 0)),
        ],
        out_specs=pl.BlockSpec((T, D), lambda e: (0, 0)),
        out_shape=jax.ShapeDtypeStruct((T, D), jnp.float32),
        compiler_params=pltpu.CompilerParams(
            dimension_semantics=("arbitrary",)),
    )(ew, yd)


def kernel(x, Wg, bg, W1, b1, W2, b2):
    x_flat = x.reshape(T, D)
    logits = x_flat @ Wg + bg
    probs = jax.nn.sigmoid(logits)
    ew = probs * (probs > THRESHOLD).astype(x_flat.dtype)
    mask = ew > 0.0
    pos = jnp.cumsum(mask.astype(jnp.int32), axis=0) - 1
    counts = jnp.sum(mask.astype(jnp.int32), axis=0)
    base = (jnp.arange(E, dtype=jnp.int32) * T)[None, :]
    posa = jnp.where(mask, pos, T - 1) + base
    posb = jnp.maximum(pos, 0) + base
    xb = x_flat.astype(jnp.bfloat16)
    xb3 = jax.lax.bitcast_convert_type(
        xb.reshape(T, D // 2, 2), jnp.int32).reshape(T, 4, 128)

    xg_i = _sc_scatter(xb3, posa.T)
    xg = jax.lax.bitcast_convert_type(
        xg_i.reshape(E, T, D // 2), jnp.bfloat16).reshape(E, T, D)
    yg = _ffn(counts, xg, W1,
              b1.reshape(E, 1, H), W2, b2.reshape(E, 1, D))
    yg3 = jax.lax.bitcast_convert_type(
        yg.reshape(E * T, D // 2, 2), jnp.int32).reshape(E * T, 4, 128)
    yd_i = _sc_gather(yg3, posb.T)
    yd = jax.lax.bitcast_convert_type(
        yd_i.reshape(E, T, D // 2), jnp.bfloat16).reshape(E, T, D)
    out = _combine(ew, yd)
    return out.reshape(B, S, D)


# speedup vs baseline: 4.5887x; 4.5887x over previous
import jax
import jax.numpy as jnp
from jax.experimental import pallas as pl
from jax.experimental.pallas import tpu as pltpu

B, S, D, H, E = 1, 2048, 1024, 4096, 8
T = B * S
THRESHOLD = 0.5

HC = 2
HB = H // HC


def _moe_kernel(ew_ref, x_ref, w1_ref, b1_ref, w2_ref, b2_ref, out_ref):
    e = pl.program_id(0)
    hc = pl.program_id(1)

    @pl.when((e == 0) & (hc == 0))
    def _init():
        out_ref[...] = jnp.zeros((T, D), jnp.float32)

    xb = x_ref[...]
    w1 = w1_ref[0].astype(jnp.bfloat16)
    h = jax.lax.dot_general(xb, w1, (((1,), (0,)), ((), ())),
                            preferred_element_type=jnp.float32)
    hb = jnp.maximum(h + b1_ref[0], 0.0).astype(jnp.bfloat16)
    w2 = w2_ref[0].astype(jnp.bfloat16)
    y = jax.lax.dot_general(hb, w2, (((1,), (0,)), ((), ())),
                            preferred_element_type=jnp.float32)
    y = y + jnp.where(hc == 0, 1.0, 0.0) * b2_ref[0]
    ewb = ew_ref[...]
    lane = jax.lax.broadcasted_iota(jnp.int32, (T, E), 1)
    w = jnp.sum(jnp.where(lane == e, ewb, 0.0), axis=1, keepdims=True)
    out_ref[...] += w * y


def kernel(x, Wg, bg, W1, b1, W2, b2):
    x_flat = x.reshape(T, D)
    logits = x_flat @ Wg + bg
    probs = jax.nn.sigmoid(logits)
    ew = probs * (probs > THRESHOLD).astype(x_flat.dtype)
    xb = x_flat.astype(jnp.bfloat16)
    b1r = b1.reshape(E, 1, H)
    b2r = b2.reshape(E, 1, D)
    out = pl.pallas_call(
        _moe_kernel,
        grid=(E, HC),
        in_specs=[
            pl.BlockSpec((T, E), lambda e, hc: (0, 0)),
            pl.BlockSpec((T, D), lambda e, hc: (0, 0)),
            pl.BlockSpec((1, D, HB), lambda e, hc: (e, 0, hc)),
            pl.BlockSpec((1, 1, HB), lambda e, hc: (e, 0, hc)),
            pl.BlockSpec((1, HB, D), lambda e, hc: (e, hc, 0)),
            pl.BlockSpec((1, 1, D), lambda e, hc: (e, 0, 0)),
        ],
        out_specs=pl.BlockSpec((T, D), lambda e, hc: (0, 0)),
        out_shape=jax.ShapeDtypeStruct((T, D), jnp.float32),
        compiler_params=pltpu.CompilerParams(
            dimension_semantics=("arbitrary", "arbitrary")),
    )(ew, xb, W1, b1r, W2, b2r)
    return out.reshape(B, S, D)
